# Initial kernel scaffold; baseline (speedup 1.0000x reference)
import functools

import jax
import jax.numpy as jnp
from jax import lax
from jax.experimental import pallas as pl
from jax.experimental.pallas import tpu as pltpu
from jax.experimental.pallas import tpu_sc as plsc

_D = 64
_M = 4
_CHUNK = 128
_LANES = 16


@functools.cache
def _make_sc_call(N, n_opc, n_opr, interpret=False):
    info = plsc.get_sparse_core_info()
    NC, NS = info.num_cores, info.num_subcores
    NW = NC * NS
    assert N % (NW * _CHUNK) == 0
    per_w = N // NW
    n_chunks = per_w // _CHUNK

    mesh = plsc.VectorSubcoreMesh(core_axis_name="c", subcore_axis_name="s")

    @functools.partial(
        pl.kernel,
        out_type=jax.ShapeDtypeStruct((N, _D), jnp.float32),
        mesh=mesh,
        interpret=interpret,
        scratch_types=[
            pltpu.VMEM((_CHUNK,), jnp.int32),
            pltpu.VMEM((_M, _CHUNK), jnp.int32),
            pltpu.VMEM((_M, _CHUNK), jnp.float32),
            pltpu.VMEM((_M, _CHUNK, _D), jnp.float32),
            pltpu.VMEM((_CHUNK, _D), jnp.float32),
            pltpu.SemaphoreType.DMA,
            pltpu.SemaphoreType.DMA,
        ],
    )
    def sc_fn(opc_ids_hbm, opr_ids_hbm, opc_tab_hbm, opr_tab_hbm, out_hbm,
              opc_v, ids_v, w_v, rows_v, o_v, sem_ids, sem_g):
        wid = lax.axis_index("s") * NC + lax.axis_index("c")
        w_base = wid * per_w

        def chunk_body(c, carry):
            base = w_base + c * _CHUNK
            cps = [pltpu.async_copy(
                opc_ids_hbm.at[pl.ds(base, _CHUNK)], opc_v, sem_ids)]
            for m in range(_M):
                cps.append(pltpu.async_copy(
                    opr_ids_hbm.at[m, pl.ds(base, _CHUNK)], ids_v.at[m],
                    sem_ids))
            for cp in cps:
                cp.wait()
            gs = [pltpu.async_copy(opc_tab_hbm.at[opc_v], o_v, sem_g)]
            for m in range(_M):
                gs.append(pltpu.async_copy(
                    opr_tab_hbm.at[ids_v.at[m]], rows_v.at[m], sem_g))
            for t in range(_CHUNK // _LANES):
                sl = pl.ds(t * _LANES, _LANES)
                mk = [jnp.where(ids_v[m, sl] != 0, 1.0, 0.0) for m in range(_M)]
                cnt = mk[0] + mk[1] + mk[2] + mk[3] + 1e-10
                for m in range(_M):
                    w_v[m, sl] = mk[m] / cnt
            for g in gs:
                g.wait()

            def inst_body(i, carry2):
                ws = [w_v[m, i] for m in range(_M)]
                for dblk in range(_D // _LANES):
                    sl = pl.ds(dblk * _LANES, _LANES)
                    acc = ws[0] * rows_v[0, i, sl]
                    for m in range(1, _M):
                        acc = acc + ws[m] * rows_v[m, i, sl]
                    plsc.addupdate(o_v.at[i, sl], acc)
                return carry2

            lax.fori_loop(0, _CHUNK, inst_body, 0, unroll=2)
            pltpu.sync_copy(o_v, out_hbm.at[pl.ds(base, _CHUNK)])
            return carry

        lax.fori_loop(0, n_chunks, chunk_body, 0)

    return sc_fn


def kernel(opcode_ids, operand_ids, opcode_table, operand_table):
    B, S = opcode_ids.shape
    N = B * S
    opc_flat = opcode_ids.reshape(N).astype(jnp.int32)
    opr_t = operand_ids.reshape(N, _M).T.astype(jnp.int32)
    fn = _make_sc_call(N, opcode_table.shape[0], operand_table.shape[0])
    out = fn(opc_flat, opr_t, opcode_table, operand_table)
    return out.reshape(B, S, _D)


# speedup vs baseline: 1.8322x; 1.8322x over previous
import functools

import jax
import jax.numpy as jnp
from jax import lax
from jax.experimental import pallas as pl
from jax.experimental.pallas import tpu as pltpu
from jax.experimental.pallas import tpu_sc as plsc

_D = 64
_M = 4
_CHUNK = 128
_LANES = 16


@functools.cache
def _make_sc_call(N, n_opc, n_opr, interpret=False):
    try:
        info = plsc.get_sparse_core_info()
        NC, NS = info.num_cores, info.num_subcores
    except ValueError:
        NC, NS = 2, 16
    NW = NC * NS
    assert N % (NW * _CHUNK) == 0
    per_w = N // NW
    n_chunks = per_w // _CHUNK

    mesh = plsc.VectorSubcoreMesh(
        core_axis_name="c", subcore_axis_name="s",
        num_cores=NC, num_subcores=NS)

    @functools.partial(
        pl.kernel,
        out_type=jax.ShapeDtypeStruct((N, _D), jnp.float32),
        mesh=mesh,
        interpret=interpret,
        compiler_params=pltpu.CompilerParams(use_tc_tiling_on_sc=False),
        scratch_types=[
            pltpu.VMEM((_CHUNK,), jnp.int32),
            pltpu.VMEM((_M, _CHUNK), jnp.int32),
            pltpu.VMEM((_M, _CHUNK), jnp.float32),
            pltpu.VMEM((_M, _CHUNK, _D), jnp.float32),
            pltpu.VMEM((_CHUNK, _D), jnp.float32),
            pltpu.SemaphoreType.DMA,
            pltpu.SemaphoreType.DMA,
        ],
    )
    def sc_fn(opc_ids_hbm, opr_ids_hbm, opc_tab_hbm, opr_tab_hbm, out_hbm,
              opc_v, ids_v, w_v, rows_v, o_v, sem_ids, sem_g):
        wid = lax.axis_index("s") * NC + lax.axis_index("c")
        w_base = wid * per_w

        def chunk_body(c, carry):
            base = w_base + c * _CHUNK
            cps = [pltpu.async_copy(
                opc_ids_hbm.at[pl.ds(base, _CHUNK)], opc_v, sem_ids)]
            for m in range(_M):
                cps.append(pltpu.async_copy(
                    opr_ids_hbm.at[m, pl.ds(base, _CHUNK)], ids_v.at[m],
                    sem_ids))
            for cp in cps:
                cp.wait()
            gs = [pltpu.async_copy(opc_tab_hbm.at[opc_v], o_v, sem_g)]
            for m in range(_M):
                gs.append(pltpu.async_copy(
                    opr_tab_hbm.at[ids_v.at[m]], rows_v.at[m], sem_g))
            for t in range(_CHUNK // _LANES):
                sl = pl.ds(t * _LANES, _LANES)
                mk = [jnp.where(ids_v[m, sl] != 0, 1.0, 0.0) for m in range(_M)]
                cnt = mk[0] + mk[1] + mk[2] + mk[3] + 1e-10
                for m in range(_M):
                    w_v[m, sl] = mk[m] / cnt
            for g in gs:
                g.wait()

            def group_body(g, carry2):
                i0 = g * _LANES
                wvecs = [w_v[m, pl.ds(i0, _LANES)] for m in range(_M)]
                for j in range(_LANES):
                    i = i0 + j
                    ws = [wvecs[m][j] for m in range(_M)]
                    for dblk in range(_D // _LANES):
                        sl = pl.ds(dblk * _LANES, _LANES)
                        acc = ws[0] * rows_v[0, i, sl]
                        for m in range(1, _M):
                            acc = acc + ws[m] * rows_v[m, i, sl]
                        plsc.addupdate(o_v.at[i, sl], acc)
                return carry2

            lax.fori_loop(0, _CHUNK // _LANES, group_body, 0)
            pltpu.sync_copy(o_v, out_hbm.at[pl.ds(base, _CHUNK)])
            return carry

        lax.fori_loop(0, n_chunks, chunk_body, 0)

    return sc_fn


def kernel(opcode_ids, operand_ids, opcode_table, operand_table):
    B, S = opcode_ids.shape
    N = B * S
    opc_flat = opcode_ids.reshape(N).astype(jnp.int32)
    opr_t = operand_ids.reshape(N, _M).T.astype(jnp.int32)
    fn = _make_sc_call(N, opcode_table.shape[0], operand_table.shape[0])
    out = fn(opc_flat, opr_t, opcode_table, operand_table)
    return out.reshape(B, S, _D)
